# baseline (speedup 1.0000x reference)
import functools

import jax
import jax.numpy as jnp
from jax import lax
from jax.experimental import pallas as pl
from jax.experimental.pallas import tpu as pltpu
from jax.experimental.pallas import tpu_sc as plsc

VOCAB = 1000000
DIM = 16
B = 4096
WINDOW = 5
CLUSTERS = 20
NEG = 10
LAMBD = 0.0625

N = B * WINDOW
NWORK = 32
CHUNK = N // NWORK
ECHUNK = B // NWORK
GCH = 128
VMAIN = 999936
VPAD = 1000064


DBLK = 249984


def _detile_body(tabT_ref, tail_ref, flat_ref, sem):
    i = pl.program_id(0)
    copies = [pltpu.make_async_copy(
        tabT_ref.at[f], flat_ref.at[pl.ds(f * VPAD + i * DBLK, DBLK)], sem)
        for f in range(DIM)]
    for c in copies:
        c.start()

    @pl.when(i == 0)
    def _():
        tails = [pltpu.make_async_copy(
            tail_ref.at[f], flat_ref.at[pl.ds(f * VPAD + VMAIN, 128)], sem)
            for f in range(DIM)]
        for c in tails:
            c.start()
        for c in tails:
            c.wait()

    for c in copies:
        c.wait()


def _detile_call(tabT, tail):
    return pl.pallas_call(
        _detile_body,
        grid=(VMAIN // DBLK,),
        in_specs=[pl.BlockSpec((DIM, DBLK), lambda i: (0, i)),
                  pl.BlockSpec((DIM, 128), lambda i: (0, 0))],
        out_specs=pl.BlockSpec(memory_space=pltpu.MemorySpace.HBM),
        out_shape=jax.ShapeDtypeStruct((DIM * VPAD,), jnp.float32),
        scratch_shapes=[pltpu.SemaphoreType.DMA],
    )(tabT, tail)


@functools.cache
def _make_emb_kernel():
    mesh = plsc.VectorSubcoreMesh(core_axis_name="c", subcore_axis_name="s")

    @functools.partial(
        pl.kernel,
        mesh=mesh,
        compiler_params=pltpu.CompilerParams(use_tc_tiling_on_sc=False),
        out_type=[
            jax.ShapeDtypeStruct((DIM, N), jnp.float32),
            jax.ShapeDtypeStruct((DIM, B), jnp.float32),
            jax.ShapeDtypeStruct((DIM, B), jnp.float32),
        ],
        scratch_types=[
            pltpu.VMEM((CHUNK,), jnp.int32),
            pltpu.VMEM((DIM, CHUNK), jnp.float32),
            pltpu.VMEM((ECHUNK,), jnp.int32),
            pltpu.VMEM((ECHUNK,), jnp.int32),
            pltpu.SemaphoreType.DMA,
        ],
    )
    def _emb_kernel(inputs_flat, el_idx, er_idx, train_inputs, emb_flat,
                    emb_out, l_out, r_out,
                    idx_v, cols_v, eidx_v, tin_v, sem):
        wid = lax.axis_index("s") * 2 + lax.axis_index("c")
        base = wid * CHUNK

        def plane_gather(tabT, idx_ref, n, cols):
            copies = []
            for f in range(DIM):
                plane = tabT.at[pl.ds(f * VPAD, VPAD)]
                dst = cols.at[f]
                for j in range(n // GCH):
                    copies.append(pltpu.async_copy(
                        plane.at[idx_ref.at[pl.ds(j * GCH, GCH)]],
                        dst.at[pl.ds(j * GCH, GCH)], sem))
            for c in copies:
                c.wait()

        pltpu.sync_copy(inputs_flat.at[pl.ds(base, CHUNK)], idx_v)
        plane_gather(emb_flat, idx_v, CHUNK, cols_v)
        pltpu.sync_copy(cols_v, emb_out.at[:, pl.ds(base, CHUNK)])

        ebase = wid * ECHUNK
        for src_, dst in ((el_idx, l_out), (er_idx, r_out)):
            pltpu.sync_copy(src_.at[pl.ds(ebase, ECHUNK)], eidx_v)
            pltpu.async_copy(train_inputs.at[eidx_v], tin_v, sem).wait()
            plane_gather(emb_flat, tin_v, ECHUNK, cols_v)
            pltpu.sync_copy(cols_v.at[:, pl.ds(0, ECHUNK)],
                            dst.at[:, pl.ds(ebase, ECHUNK)])

    return _emb_kernel


@functools.cache
def _make_nce_kernel():
    mesh = plsc.VectorSubcoreMesh(core_axis_name="c", subcore_axis_name="s")

    @functools.partial(
        pl.kernel,
        mesh=mesh,
        compiler_params=pltpu.CompilerParams(use_tc_tiling_on_sc=False),
        out_type=[
            jax.ShapeDtypeStruct((DIM, N), jnp.float32),
            jax.ShapeDtypeStruct((N,), jnp.float32),
            jax.ShapeDtypeStruct((DIM, 16), jnp.float32),
            jax.ShapeDtypeStruct((16,), jnp.float32),
        ],
        scratch_types=[
            pltpu.VMEM((CHUNK,), jnp.int32),
            pltpu.VMEM((DIM, CHUNK), jnp.float32),
            pltpu.VMEM((CHUNK,), jnp.float32),
            pltpu.VMEM((16,), jnp.int32),
            pltpu.SemaphoreType.DMA,
        ],
    )
    def _nce_kernel(labels_flat, samp_ids, nce_flat, nce_b,
                    tw_out, tb_out, sw_out, sb_out,
                    idx_v, cols_v, b_v, sidx_v, sem):
        wid = lax.axis_index("s") * 2 + lax.axis_index("c")
        base = wid * CHUNK

        def plane_gather(tabT, idx_ref, n, cols):
            copies = []
            for f in range(DIM):
                plane = tabT.at[pl.ds(f * VPAD, VPAD)]
                dst = cols.at[f]
                for j in range(n // GCH):
                    copies.append(pltpu.async_copy(
                        plane.at[idx_ref.at[pl.ds(j * GCH, GCH)]],
                        dst.at[pl.ds(j * GCH, GCH)], sem))
            for c in copies:
                c.wait()

        pltpu.sync_copy(labels_flat.at[pl.ds(base, CHUNK)], idx_v)
        plane_gather(nce_flat, idx_v, CHUNK, cols_v)
        pltpu.sync_copy(cols_v, tw_out.at[:, pl.ds(base, CHUNK)])
        for j in range(CHUNK // GCH):
            pltpu.async_copy(nce_b.at[idx_v.at[pl.ds(j * GCH, GCH)]],
                             b_v.at[pl.ds(j * GCH, GCH)], sem).wait()
        pltpu.sync_copy(b_v, tb_out.at[pl.ds(base, CHUNK)])

        @pl.when(wid == 0)
        def _():
            pltpu.sync_copy(samp_ids, sidx_v)
            copies = []
            for f in range(DIM):
                copies.append(pltpu.async_copy(
                    nce_flat.at[pl.ds(f * VPAD, VPAD)].at[sidx_v],
                    cols_v.at[f].at[pl.ds(0, 16)], sem))
            for c in copies:
                c.wait()
            pltpu.sync_copy(cols_v.at[:, pl.ds(0, 16)], sw_out)
            pltpu.async_copy(nce_b.at[sidx_v], b_v.at[pl.ds(0, 16)], sem).wait()
            pltpu.sync_copy(b_v.at[pl.ds(0, 16)], sb_out)

    return _nce_kernel


def _clip_t(x):
    n = jnp.sqrt(jnp.sum(x * x, axis=0, keepdims=True))
    scale = jnp.where(n > 1.0, 1.0 / jnp.maximum(n, 1e-12), 1.0)
    return x * scale


def _dense_body(embT_ref, twT_ref, tb_ref, swT_ref, sb_ref, lT_ref, rT_ref,
                ov_ref, nzT_ref, cm_ref, g_ref, o_ref):
    embT = _clip_t(embT_ref[...])
    twT = twT_ref[...]

    true_l = jnp.sum(embT * twT, axis=0, keepdims=True) + tb_ref[...]
    sl = lax.dot_general(swT_ref[...], embT, (((0,), (0,)), ((), ())),
                         preferred_element_type=jnp.float32)
    sl = sl + sb_ref[...]
    row = lax.broadcasted_iota(jnp.int32, (16, N), 0)
    slm = jnp.where(row < NEG, sl, -1e30)
    m = jnp.maximum(true_l, jnp.max(slm, axis=0, keepdims=True))
    se = jnp.exp(true_l - m) + jnp.sum(jnp.exp(slm - m), axis=0, keepdims=True)
    per_ex = jnp.log(se) + m - true_l
    emb_loss = jnp.sum(per_ex) * (1.0 / N)

    cm = cm_ref[...]
    dots = jnp.dot(cm, embT, preferred_element_type=jnp.float32)
    c2 = jnp.sum(cm * cm, axis=1, keepdims=True)
    e2 = jnp.sum(embT * embT, axis=0, keepdims=True)
    d2 = e2 - 2.0 * dots + c2
    dist = jnp.sqrt(jnp.maximum(d2, 0.0) + 1e-12)
    clus_loss = jnp.sum(jnp.min(dist, axis=0, keepdims=True)) * (1.0 / N)

    diff = _clip_t(_clip_t(lT_ref[...])) - _clip_t(_clip_t(rT_ref[...])) + nzT_ref[...]
    rd = jnp.sqrt(jnp.sum(diff * diff, axis=0, keepdims=True) + 1e-12)
    reg_loss = jnp.sum(ov_ref[...] * rd)

    total = emb_loss + g_ref[0, 0] * clus_loss + LAMBD * reg_loss
    o_ref[...] = jnp.broadcast_to(total, (1, 1))


def _dense_call(embT, twT, tb_row, swT, sb, lT, rT, ov_row, nzT, cm_pad, g2,
                interpret=False):
    return pl.pallas_call(
        _dense_body,
        out_shape=jax.ShapeDtypeStruct((1, 1), jnp.float32),
        interpret=interpret,
    )(embT, twT, tb_row, swT, sb, lT, rT, ov_row, nzT, cm_pad, g2)


def kernel(train_inputs, train_labels, edge_indices_left, edge_indices_right,
           overlap, sampled_ids, gamma, embedding_matrix, nce_weights,
           nce_biases, cluster_means, noise):
    labels_flat = train_labels.reshape(-1)
    inputs_flat = jnp.repeat(train_inputs, WINDOW)
    el = jnp.concatenate([edge_indices_left // WINDOW,
                          jnp.zeros((1,), jnp.int32)])
    er = jnp.concatenate([edge_indices_right // WINDOW,
                          jnp.zeros((1,), jnp.int32)])
    samp = jnp.concatenate([sampled_ids, jnp.zeros((16 - NEG,), jnp.int32)])

    emb_tail = jnp.pad(embedding_matrix[VMAIN:], ((0, 128 - (VOCAB - VMAIN)), (0, 0))).T
    nce_tail = jnp.pad(nce_weights[VMAIN:], ((0, 128 - (VOCAB - VMAIN)), (0, 0))).T
    nce_flat = _detile_call(nce_weights.T, nce_tail)
    twT_r, tb_r, swT, sb = _make_nce_kernel()(labels_flat, samp, nce_flat,
                                              nce_biases)
    emb_flat = _detile_call(embedding_matrix.T, emb_tail)
    embT_r, lT_r, rT_r = _make_emb_kernel()(inputs_flat, el, er, train_inputs,
                                            emb_flat)
    nzT = jnp.concatenate([noise, jnp.zeros((1, DIM), jnp.float32)], 0).T
    ov_row = jnp.concatenate([overlap, jnp.zeros((1, 1), jnp.float32)],
                             0).reshape(1, B)
    cm_pad = jnp.concatenate(
        [cluster_means, jnp.full((32 - CLUSTERS, DIM), 1e3, jnp.float32)], 0)

    out = _dense_call(embT_r, twT_r, tb_r.reshape(1, N), swT, sb.reshape(16, 1),
                      lT_r, rT_r, ov_row, nzT, cm_pad, gamma.reshape(1, 1))
    return out[0, 0]


# speedup vs baseline: 1.0134x; 1.0134x over previous
import functools

import jax
import jax.numpy as jnp
from jax import lax
from jax.experimental import pallas as pl
from jax.experimental.pallas import tpu as pltpu
from jax.experimental.pallas import tpu_sc as plsc

VOCAB = 1000000
DIM = 16
B = 4096
WINDOW = 5
CLUSTERS = 20
NEG = 10
LAMBD = 0.0625

N = B * WINDOW
NWORK = 32
CHUNK = N // NWORK
ECHUNK = B // NWORK
GCH = 128
VMAIN = 999936
VPAD = 1000064


DBLK = 166656


def _detile_body(tabT_ref, tail_ref, flat_ref, sem):
    i = pl.program_id(0)
    copies = [pltpu.make_async_copy(
        tabT_ref.at[f], flat_ref.at[pl.ds(f * VPAD + i * DBLK, DBLK)], sem)
        for f in range(DIM)]
    for c in copies:
        c.start()

    @pl.when(i == 0)
    def _():
        tails = [pltpu.make_async_copy(
            tail_ref.at[f], flat_ref.at[pl.ds(f * VPAD + VMAIN, 128)], sem)
            for f in range(DIM)]
        for c in tails:
            c.start()
        for c in tails:
            c.wait()

    for c in copies:
        c.wait()


def _detile_call(tabT, tail):
    return pl.pallas_call(
        _detile_body,
        grid=(VMAIN // DBLK,),
        in_specs=[pl.BlockSpec((DIM, DBLK), lambda i: (0, i)),
                  pl.BlockSpec((DIM, 128), lambda i: (0, 0))],
        out_specs=pl.BlockSpec(memory_space=pltpu.MemorySpace.HBM),
        out_shape=jax.ShapeDtypeStruct((DIM * VPAD,), jnp.float32),
        scratch_shapes=[pltpu.SemaphoreType.DMA],
    )(tabT, tail)


@functools.cache
def _make_emb_kernel():
    mesh = plsc.VectorSubcoreMesh(core_axis_name="c", subcore_axis_name="s")

    @functools.partial(
        pl.kernel,
        mesh=mesh,
        compiler_params=pltpu.CompilerParams(use_tc_tiling_on_sc=False),
        out_type=[
            jax.ShapeDtypeStruct((DIM, N), jnp.float32),
            jax.ShapeDtypeStruct((DIM, B), jnp.float32),
            jax.ShapeDtypeStruct((DIM, B), jnp.float32),
        ],
        scratch_types=[
            pltpu.VMEM((CHUNK,), jnp.int32),
            pltpu.VMEM((DIM, CHUNK), jnp.float32),
            pltpu.VMEM((ECHUNK,), jnp.int32),
            pltpu.VMEM((ECHUNK,), jnp.int32),
            pltpu.SemaphoreType.DMA,
        ],
    )
    def _emb_kernel(inputs_flat, el_idx, er_idx, train_inputs, emb_flat,
                    emb_out, l_out, r_out,
                    idx_v, cols_v, eidx_v, tin_v, sem):
        wid = lax.axis_index("s") * 2 + lax.axis_index("c")
        base = wid * CHUNK

        def plane_gather(tabT, idx_ref, n, cols):
            copies = []
            for f in range(DIM):
                plane = tabT.at[pl.ds(f * VPAD, VPAD)]
                dst = cols.at[f]
                for j in range(n // GCH):
                    copies.append(pltpu.async_copy(
                        plane.at[idx_ref.at[pl.ds(j * GCH, GCH)]],
                        dst.at[pl.ds(j * GCH, GCH)], sem))
            for c in copies:
                c.wait()

        pltpu.sync_copy(inputs_flat.at[pl.ds(base, CHUNK)], idx_v)
        plane_gather(emb_flat, idx_v, CHUNK, cols_v)
        pltpu.sync_copy(cols_v, emb_out.at[:, pl.ds(base, CHUNK)])

        ebase = wid * ECHUNK
        for src_, dst in ((el_idx, l_out), (er_idx, r_out)):
            pltpu.sync_copy(src_.at[pl.ds(ebase, ECHUNK)], eidx_v)
            pltpu.async_copy(train_inputs.at[eidx_v], tin_v, sem).wait()
            plane_gather(emb_flat, tin_v, ECHUNK, cols_v)
            pltpu.sync_copy(cols_v.at[:, pl.ds(0, ECHUNK)],
                            dst.at[:, pl.ds(ebase, ECHUNK)])

    return _emb_kernel


@functools.cache
def _make_nce_kernel():
    mesh = plsc.VectorSubcoreMesh(core_axis_name="c", subcore_axis_name="s")

    @functools.partial(
        pl.kernel,
        mesh=mesh,
        compiler_params=pltpu.CompilerParams(use_tc_tiling_on_sc=False),
        out_type=[
            jax.ShapeDtypeStruct((DIM, N), jnp.float32),
            jax.ShapeDtypeStruct((N,), jnp.float32),
            jax.ShapeDtypeStruct((DIM, 16), jnp.float32),
            jax.ShapeDtypeStruct((16,), jnp.float32),
        ],
        scratch_types=[
            pltpu.VMEM((CHUNK,), jnp.int32),
            pltpu.VMEM((DIM, CHUNK), jnp.float32),
            pltpu.VMEM((CHUNK,), jnp.float32),
            pltpu.VMEM((16,), jnp.int32),
            pltpu.SemaphoreType.DMA,
        ],
    )
    def _nce_kernel(labels_flat, samp_ids, nce_flat, nce_b,
                    tw_out, tb_out, sw_out, sb_out,
                    idx_v, cols_v, b_v, sidx_v, sem):
        wid = lax.axis_index("s") * 2 + lax.axis_index("c")
        base = wid * CHUNK

        def plane_gather(tabT, idx_ref, n, cols):
            copies = []
            for f in range(DIM):
                plane = tabT.at[pl.ds(f * VPAD, VPAD)]
                dst = cols.at[f]
                for j in range(n // GCH):
                    copies.append(pltpu.async_copy(
                        plane.at[idx_ref.at[pl.ds(j * GCH, GCH)]],
                        dst.at[pl.ds(j * GCH, GCH)], sem))
            for c in copies:
                c.wait()

        pltpu.sync_copy(labels_flat.at[pl.ds(base, CHUNK)], idx_v)
        plane_gather(nce_flat, idx_v, CHUNK, cols_v)
        pltpu.sync_copy(cols_v, tw_out.at[:, pl.ds(base, CHUNK)])
        for j in range(CHUNK // GCH):
            pltpu.async_copy(nce_b.at[idx_v.at[pl.ds(j * GCH, GCH)]],
                             b_v.at[pl.ds(j * GCH, GCH)], sem).wait()
        pltpu.sync_copy(b_v, tb_out.at[pl.ds(base, CHUNK)])

        @pl.when(wid == 0)
        def _():
            pltpu.sync_copy(samp_ids, sidx_v)
            copies = []
            for f in range(DIM):
                copies.append(pltpu.async_copy(
                    nce_flat.at[pl.ds(f * VPAD, VPAD)].at[sidx_v],
                    cols_v.at[f].at[pl.ds(0, 16)], sem))
            for c in copies:
                c.wait()
            pltpu.sync_copy(cols_v.at[:, pl.ds(0, 16)], sw_out)
            pltpu.async_copy(nce_b.at[sidx_v], b_v.at[pl.ds(0, 16)], sem).wait()
            pltpu.sync_copy(b_v.at[pl.ds(0, 16)], sb_out)

    return _nce_kernel


def _clip_t(x):
    n = jnp.sqrt(jnp.sum(x * x, axis=0, keepdims=True))
    scale = jnp.where(n > 1.0, 1.0 / jnp.maximum(n, 1e-12), 1.0)
    return x * scale


def _dense_body(embT_ref, twT_ref, tb_ref, swT_ref, sb_ref, lT_ref, rT_ref,
                ov_ref, nzT_ref, cm_ref, g_ref, o_ref):
    embT = _clip_t(embT_ref[...])
    twT = twT_ref[...]

    true_l = jnp.sum(embT * twT, axis=0, keepdims=True) + tb_ref[...]
    sl = lax.dot_general(swT_ref[...], embT, (((0,), (0,)), ((), ())),
                         preferred_element_type=jnp.float32)
    sl = sl + sb_ref[...]
    row = lax.broadcasted_iota(jnp.int32, (16, N), 0)
    slm = jnp.where(row < NEG, sl, -1e30)
    m = jnp.maximum(true_l, jnp.max(slm, axis=0, keepdims=True))
    se = jnp.exp(true_l - m) + jnp.sum(jnp.exp(slm - m), axis=0, keepdims=True)
    per_ex = jnp.log(se) + m - true_l
    emb_loss = jnp.sum(per_ex) * (1.0 / N)

    cm = cm_ref[...]
    dots = jnp.dot(cm, embT, preferred_element_type=jnp.float32)
    c2 = jnp.sum(cm * cm, axis=1, keepdims=True)
    e2 = jnp.sum(embT * embT, axis=0, keepdims=True)
    d2 = e2 - 2.0 * dots + c2
    dist = jnp.sqrt(jnp.maximum(d2, 0.0) + 1e-12)
    clus_loss = jnp.sum(jnp.min(dist, axis=0, keepdims=True)) * (1.0 / N)

    diff = _clip_t(_clip_t(lT_ref[...])) - _clip_t(_clip_t(rT_ref[...])) + nzT_ref[...]
    rd = jnp.sqrt(jnp.sum(diff * diff, axis=0, keepdims=True) + 1e-12)
    reg_loss = jnp.sum(ov_ref[...] * rd)

    total = emb_loss + g_ref[0, 0] * clus_loss + LAMBD * reg_loss
    o_ref[...] = jnp.broadcast_to(total, (1, 1))


def _dense_call(embT, twT, tb_row, swT, sb, lT, rT, ov_row, nzT, cm_pad, g2,
                interpret=False):
    return pl.pallas_call(
        _dense_body,
        out_shape=jax.ShapeDtypeStruct((1, 1), jnp.float32),
        interpret=interpret,
    )(embT, twT, tb_row, swT, sb, lT, rT, ov_row, nzT, cm_pad, g2)


def kernel(train_inputs, train_labels, edge_indices_left, edge_indices_right,
           overlap, sampled_ids, gamma, embedding_matrix, nce_weights,
           nce_biases, cluster_means, noise):
    labels_flat = train_labels.reshape(-1)
    inputs_flat = jnp.repeat(train_inputs, WINDOW)
    el = jnp.concatenate([edge_indices_left // WINDOW,
                          jnp.zeros((1,), jnp.int32)])
    er = jnp.concatenate([edge_indices_right // WINDOW,
                          jnp.zeros((1,), jnp.int32)])
    samp = jnp.concatenate([sampled_ids, jnp.zeros((16 - NEG,), jnp.int32)])

    emb_tail = jnp.pad(embedding_matrix[VMAIN:], ((0, 128 - (VOCAB - VMAIN)), (0, 0))).T
    nce_tail = jnp.pad(nce_weights[VMAIN:], ((0, 128 - (VOCAB - VMAIN)), (0, 0))).T
    nce_flat = _detile_call(nce_weights.T, nce_tail)
    twT_r, tb_r, swT, sb = _make_nce_kernel()(labels_flat, samp, nce_flat,
                                              nce_biases)
    emb_flat = _detile_call(embedding_matrix.T, emb_tail)
    embT_r, lT_r, rT_r = _make_emb_kernel()(inputs_flat, el, er, train_inputs,
                                            emb_flat)
    nzT = jnp.concatenate([noise, jnp.zeros((1, DIM), jnp.float32)], 0).T
    ov_row = jnp.concatenate([overlap, jnp.zeros((1, 1), jnp.float32)],
                             0).reshape(1, B)
    cm_pad = jnp.concatenate(
        [cluster_means, jnp.full((32 - CLUSTERS, DIM), 1e3, jnp.float32)], 0)

    out = _dense_call(embT_r, twT_r, tb_r.reshape(1, N), swT, sb.reshape(16, 1),
                      lT_r, rT_r, ov_row, nzT, cm_pad, gamma.reshape(1, 1))
    return out[0, 0]
